# baseline (speedup 1.0000x reference)
import functools
import jax
import jax.numpy as jnp
from jax import lax
from jax.experimental import pallas as pl
from jax.experimental.pallas import tpu as pltpu
from jax.experimental.pallas import tpu_sc as plsc

_N_EXP = 2
_D = 1024
_NA = 16
_TILE = 256
_N = 2048
_NT = _N // _TILE + 1
_NPAD = _NT * _TILE
_SMALL = 128
_NW = 32
_CHUNK = _N // _NW


def _route_body(f_ref, dest_ref, te_ref):
    f = f_ref[...]
    rows, cols = 8, 256
    ki = lax.broadcasted_iota(jnp.int32, (cols, cols), 0)
    kj = lax.broadcasted_iota(jnp.int32, (cols, cols), 1)
    upper = (ki <= kj).astype(jnp.float32)
    ones_cs = jnp.dot(f, upper, preferred_element_type=jnp.float32)
    row_tot = ones_cs[:, cols - 1:cols]
    li = lax.broadcasted_iota(jnp.int32, (rows, rows), 0)
    lj = lax.broadcasted_iota(jnp.int32, (rows, rows), 1)
    lower = (lj < li).astype(jnp.float32)
    offs = jnp.dot(lower, row_tot, preferred_element_type=jnp.float32)
    ones_excl = offs + ones_cs - f
    pi = lax.broadcasted_iota(jnp.int32, (rows, cols), 0)
    pj = lax.broadcasted_iota(jnp.int32, (rows, cols), 1)
    pos = (pi * cols + pj).astype(jnp.float32)
    zeros_excl = pos - ones_excl
    c1 = jnp.sum(f)
    c0 = jnp.float32(_N) - c1
    n0t = jnp.floor((c0 + jnp.float32(_TILE - 1)) / jnp.float32(_TILE))
    start1 = n0t * jnp.float32(_TILE)
    dest = jnp.where(f == 1.0, start1 + ones_excl, zeros_excl)
    dest_ref[...] = dest.astype(jnp.int32)
    tj = lax.broadcasted_iota(jnp.int32, (rows, _SMALL), 1)
    te_ref[...] = (tj.astype(jnp.float32) >= n0t).astype(jnp.float32)


def _route(instrf2):
    return pl.pallas_call(
        _route_body,
        out_shape=[
            jax.ShapeDtypeStruct((8, 256), jnp.int32),
            jax.ShapeDtypeStruct((8, _SMALL), jnp.float32),
        ],
    )(instrf2)


def _sc_mesh():
    return plsc.VectorSubcoreMesh(core_axis_name="c", subcore_axis_name="s",
                                  num_cores=2, num_subcores=16)


@functools.lru_cache(maxsize=None)
def _make_sc_scatter_fwd():
    @functools.partial(
        pl.kernel,
        mesh=_sc_mesh(),
        out_type=[
            jax.ShapeDtypeStruct((_NPAD, _D), jnp.float32),
            jax.ShapeDtypeStruct((_NPAD, _D), jnp.float32),
        ],
        scratch_types=[
            pltpu.VMEM((_CHUNK,), jnp.int32),
            pltpu.VMEM((_CHUNK, _D), jnp.float32),
            pltpu.SemaphoreType.DMA,
        ],
    )
    def _sc_scatter_fwd(dest_hbm, base_hbm, rnn_hbm, baseg_hbm, rnng_hbm,
                        dest_v, rowbuf, sem):
        wid = lax.axis_index("s") * 2 + lax.axis_index("c")
        off = wid * _CHUNK
        pltpu.sync_copy(dest_hbm.at[pl.ds(off, _CHUNK)], dest_v)
        pltpu.sync_copy(base_hbm.at[pl.ds(off, _CHUNK)], rowbuf)
        pltpu.async_copy(rowbuf, baseg_hbm.at[dest_v], sem).wait()
        pltpu.sync_copy(rnn_hbm.at[pl.ds(off, _CHUNK)], rowbuf)
        pltpu.async_copy(rowbuf, rnng_hbm.at[dest_v], sem).wait()

    return _sc_scatter_fwd


def _compute_body(te_ref, x_ref, r_ref, w1_ref, wh_ref, b1_ref, h_out_ref):
    x = x_ref[...]
    r = r_ref[...]
    pre = (jnp.dot(x, w1_ref[0], preferred_element_type=jnp.float32)
           + jnp.dot(r, wh_ref[0], preferred_element_type=jnp.float32)
           + b1_ref[0])
    h_out_ref[...] = jnp.tanh(pre)


def _compute(te, base_g, rnn_g, w1, wh, b1s):
    d = _D
    grid_spec = pltpu.PrefetchScalarGridSpec(
        num_scalar_prefetch=1,
        grid=(_NT,),
        in_specs=[
            pl.BlockSpec((_TILE, d), lambda t, te: (t, 0)),
            pl.BlockSpec((_TILE, d), lambda t, te: (t, 0)),
            pl.BlockSpec((1, d, d), lambda t, te: (te[t], 0, 0)),
            pl.BlockSpec((1, d, d), lambda t, te: (te[t], 0, 0)),
            pl.BlockSpec((1, 1, d), lambda t, te: (te[t], 0, 0)),
        ],
        out_specs=[
            pl.BlockSpec((_TILE, d), lambda t, te: (t, 0)),
        ],
    )
    return pl.pallas_call(
        _compute_body,
        grid_spec=grid_spec,
        out_shape=[
            jax.ShapeDtypeStruct((_NPAD, d), jnp.float32),
        ],
        compiler_params=pltpu.CompilerParams(
            dimension_semantics=("arbitrary",),
        ),
    )(te, base_g, rnn_g, w1, wh, b1s)[0]


def _head_body(instr_ref, h_ref, wc_ref, bc_ref, small_out_ref):
    h = h_ref[...]
    col = lax.broadcasted_iota(jnp.int32, (_TILE, _SMALL), 1)
    is_logit = col < _NA
    for e in range(_N_EXP):
        head = (jnp.dot(h, wc_ref[e], preferred_element_type=jnp.float32)
                + bc_ref[e])
        ml = jnp.where(is_logit, head, jnp.float32(-1e30))
        m = jnp.max(ml, axis=1, keepdims=True)
        amax = jnp.min(jnp.where((ml == m) & is_logit, col, _SMALL),
                       axis=1, keepdims=True).astype(jnp.float32)
        se = jnp.sum(jnp.where(is_logit, jnp.exp(ml - m), 0.0),
                     axis=1, keepdims=True)
        lp = -jnp.log(se)
        v = jnp.sum(jnp.where(col == _NA, head, 0.0), axis=1, keepdims=True)
        small = jnp.where(col == 0, v,
                          jnp.where(col == 1, amax,
                                    jnp.where(col == 2, lp, 0.0)))
        flag = instr_ref[...] == jnp.float32(e)
        if e == 0:
            small_out_ref[...] = jnp.where(flag, small, 0.0)
        else:
            small_out_ref[...] = jnp.where(flag, small, small_out_ref[...])


def _head(instrf, hxs, wc, bc):
    d = _D
    return pl.pallas_call(
        _head_body,
        grid=(_N // _TILE,),
        in_specs=[
            pl.BlockSpec((_TILE, 1), lambda t: (t, 0)),
            pl.BlockSpec((_TILE, d), lambda t: (t, 0)),
            pl.BlockSpec((_N_EXP, d, _SMALL), lambda t: (0, 0, 0)),
            pl.BlockSpec((_N_EXP, 1, _SMALL), lambda t: (0, 0, 0)),
        ],
        out_specs=[
            pl.BlockSpec((_TILE, _SMALL), lambda t: (t, 0)),
        ],
        out_shape=[
            jax.ShapeDtypeStruct((_N, _SMALL), jnp.float32),
        ],
        compiler_params=pltpu.CompilerParams(
            dimension_semantics=("arbitrary",),
        ),
    )(instrf, hxs, wc, bc)[0]


@functools.lru_cache(maxsize=None)
def _make_sc_gather_back():
    @functools.partial(
        pl.kernel,
        mesh=_sc_mesh(),
        out_type=[
            jax.ShapeDtypeStruct((_N, _D), jnp.float32),
        ],
        scratch_types=[
            pltpu.VMEM((_CHUNK,), jnp.int32),
            pltpu.VMEM((_CHUNK, _D), jnp.float32),
            pltpu.SemaphoreType.DMA,
        ],
    )
    def _sc_gather_back(dest_hbm, hg_hbm, h_hbm, dest_v, rowbuf, sem):
        wid = lax.axis_index("s") * 2 + lax.axis_index("c")
        off = wid * _CHUNK
        pltpu.sync_copy(dest_hbm.at[pl.ds(off, _CHUNK)], dest_v)
        pltpu.async_copy(hg_hbm.at[dest_v], rowbuf, sem).wait()
        pltpu.sync_copy(rowbuf, h_hbm.at[pl.ds(off, _CHUNK)])

    return _sc_gather_back


def kernel(base, instructions, rnn_hxs, masks, W1, b1, Wh, Wv, bv, Wa, ba):
    n, d = base.shape
    rnnm = rnn_hxs * masks
    instrf2 = instructions.astype(jnp.float32).reshape(8, 256)
    w1 = W1[:_N_EXP]
    wh = Wh[:_N_EXP]
    b1s = b1[:_N_EXP]
    pad = jnp.zeros((_N_EXP, d, _SMALL - _NA - 1), dtype=jnp.float32)
    wc = jnp.concatenate([Wa[:_N_EXP], Wv[:_N_EXP], pad], axis=-1)
    bpad = jnp.zeros((_N_EXP, _SMALL - _NA - 1), dtype=jnp.float32)
    bc = jnp.concatenate([ba[:_N_EXP], bv[:_N_EXP], bpad], axis=-1)

    b1s = b1s.reshape(_N_EXP, 1, d)
    bc = bc.reshape(_N_EXP, 1, _SMALL)

    dest2d, te2d = _route(instrf2)
    dest = dest2d.reshape(n)
    te = te2d[0, :_NT].astype(jnp.int32)

    base_g, rnn_g = _make_sc_scatter_fwd()(dest, base, rnnm)
    h_g = _compute(te, base_g, rnn_g, w1, wh, b1s)
    hxs = _make_sc_gather_back()(dest, h_g)[0]
    instrf = instructions.astype(jnp.float32).reshape(n, 1)
    small = _head(instrf, hxs, wc, bc)

    value = small[:, 0:1]
    action = small[:, 1:2].astype(jnp.int32)
    alp = small[:, 2:3]
    return value, action, alp, hxs


# speedup vs baseline: 1.3768x; 1.3768x over previous
import jax
import jax.numpy as jnp
from jax import lax
from jax.experimental import pallas as pl
from jax.experimental.pallas import tpu as pltpu

_N_EXP = 2
_D = 1024
_NA = 16
_TILE = 256
_SMALL = 128


def _tile_body(instr_ref, x_ref, r_ref, w1_ref, wh_ref, b1_ref,
               wc_ref, bc_ref, h_out_ref, small_out_ref):
    x = x_ref[...]
    r = r_ref[...]
    col = lax.broadcasted_iota(jnp.int32, (_TILE, _SMALL), 1)
    is_logit = col < _NA
    for e in range(_N_EXP):
        pre = (jnp.dot(x, w1_ref[e], preferred_element_type=jnp.float32)
               + jnp.dot(r, wh_ref[e], preferred_element_type=jnp.float32)
               + b1_ref[e][None, :])
        h = jnp.tanh(pre)
        head = (jnp.dot(h, wc_ref[e], preferred_element_type=jnp.float32)
                + bc_ref[e][None, :])
        ml = jnp.where(is_logit, head, jnp.float32(-1e30))
        m = jnp.max(ml, axis=1, keepdims=True)
        amax = jnp.min(jnp.where((ml == m) & is_logit, col, _SMALL),
                       axis=1, keepdims=True).astype(jnp.float32)
        se = jnp.sum(jnp.where(is_logit, jnp.exp(ml - m), 0.0),
                     axis=1, keepdims=True)
        lp = -jnp.log(se)
        v = jnp.sum(jnp.where(col == _NA, head, 0.0), axis=1, keepdims=True)
        small = jnp.where(col == 0, v,
                          jnp.where(col == 1, amax,
                                    jnp.where(col == 2, lp, 0.0)))
        flag = instr_ref[...] == jnp.float32(e)
        if e == 0:
            h_out_ref[...] = jnp.where(flag, h, 0.0)
            small_out_ref[...] = jnp.where(flag, small, 0.0)
        else:
            h_out_ref[...] = jnp.where(flag, h, h_out_ref[...])
            small_out_ref[...] = jnp.where(flag, small, small_out_ref[...])


def kernel(base, instructions, rnn_hxs, masks, W1, b1, Wh, Wv, bv, Wa, ba):
    n, d = base.shape
    rnnm = rnn_hxs * masks
    instrf = instructions.astype(jnp.float32).reshape(n, 1)
    w1 = W1[:_N_EXP]
    wh = Wh[:_N_EXP]
    b1s = b1[:_N_EXP]
    pad = jnp.zeros((_N_EXP, d, _SMALL - _NA - 1), dtype=jnp.float32)
    wc = jnp.concatenate([Wa[:_N_EXP], Wv[:_N_EXP], pad], axis=-1)
    bpad = jnp.zeros((_N_EXP, _SMALL - _NA - 1), dtype=jnp.float32)
    bc = jnp.concatenate([ba[:_N_EXP], bv[:_N_EXP], bpad], axis=-1)

    grid = (n // _TILE,)
    h_out, small_out = pl.pallas_call(
        _tile_body,
        grid=grid,
        in_specs=[
            pl.BlockSpec((_TILE, 1), lambda t: (t, 0)),
            pl.BlockSpec((_TILE, d), lambda t: (t, 0)),
            pl.BlockSpec((_TILE, d), lambda t: (t, 0)),
            pl.BlockSpec((_N_EXP, d, d), lambda t: (0, 0, 0)),
            pl.BlockSpec((_N_EXP, d, d), lambda t: (0, 0, 0)),
            pl.BlockSpec((_N_EXP, d), lambda t: (0, 0)),
            pl.BlockSpec((_N_EXP, d, _SMALL), lambda t: (0, 0, 0)),
            pl.BlockSpec((_N_EXP, _SMALL), lambda t: (0, 0)),
        ],
        out_specs=[
            pl.BlockSpec((_TILE, d), lambda t: (t, 0)),
            pl.BlockSpec((_TILE, _SMALL), lambda t: (t, 0)),
        ],
        out_shape=[
            jax.ShapeDtypeStruct((n, d), jnp.float32),
            jax.ShapeDtypeStruct((n, _SMALL), jnp.float32),
        ],
        compiler_params=pltpu.CompilerParams(
            dimension_semantics=("arbitrary",),
        ),
    )(instrf, base, rnnm, w1, wh, b1s, wc, bc)

    value = small_out[:, 0:1]
    action = small_out[:, 1:2].astype(jnp.int32)
    alp = small_out[:, 2:3]
    return value, action, alp, h_out


# speedup vs baseline: 1.3942x; 1.0126x over previous
import jax
import jax.numpy as jnp
from jax import lax
from jax.experimental import pallas as pl
from jax.experimental.pallas import tpu as pltpu

_N_EXP = 2
_D = 1024
_NA = 16
_TILE = 512
_SMALL = 128


def _tile_body(instr_ref, x_ref, r_ref, w1_ref, wh_ref, b1_ref,
               wc_ref, bc_ref, h_out_ref, small_out_ref):
    x = x_ref[...]
    r = r_ref[...]
    col = lax.broadcasted_iota(jnp.int32, (_TILE, _SMALL), 1)
    is_logit = col < _NA
    for e in range(_N_EXP):
        pre = (jnp.dot(x, w1_ref[e], preferred_element_type=jnp.float32)
               + jnp.dot(r, wh_ref[e], preferred_element_type=jnp.float32)
               + b1_ref[e][None, :])
        h = jnp.tanh(pre)
        head = (jnp.dot(h, wc_ref[e], preferred_element_type=jnp.float32)
                + bc_ref[e][None, :])
        ml = jnp.where(is_logit, head, jnp.float32(-1e30))
        m = jnp.max(ml, axis=1, keepdims=True)
        amax = jnp.min(jnp.where((ml == m) & is_logit, col, _SMALL),
                       axis=1, keepdims=True).astype(jnp.float32)
        se = jnp.sum(jnp.where(is_logit, jnp.exp(ml - m), 0.0),
                     axis=1, keepdims=True)
        lp = -jnp.log(se)
        v = jnp.sum(jnp.where(col == _NA, head, 0.0), axis=1, keepdims=True)
        small = jnp.where(col == 0, v,
                          jnp.where(col == 1, amax,
                                    jnp.where(col == 2, lp, 0.0)))
        flag = instr_ref[...] == jnp.float32(e)
        if e == 0:
            h_out_ref[...] = jnp.where(flag, h, 0.0)
            small_out_ref[...] = jnp.where(flag, small, 0.0)
        else:
            h_out_ref[...] = jnp.where(flag, h, h_out_ref[...])
            small_out_ref[...] = jnp.where(flag, small, small_out_ref[...])


def kernel(base, instructions, rnn_hxs, masks, W1, b1, Wh, Wv, bv, Wa, ba):
    n, d = base.shape
    rnnm = rnn_hxs * masks
    instrf = instructions.astype(jnp.float32).reshape(n, 1)
    w1 = W1[:_N_EXP]
    wh = Wh[:_N_EXP]
    b1s = b1[:_N_EXP]
    pad = jnp.zeros((_N_EXP, d, _SMALL - _NA - 1), dtype=jnp.float32)
    wc = jnp.concatenate([Wa[:_N_EXP], Wv[:_N_EXP], pad], axis=-1)
    bpad = jnp.zeros((_N_EXP, _SMALL - _NA - 1), dtype=jnp.float32)
    bc = jnp.concatenate([ba[:_N_EXP], bv[:_N_EXP], bpad], axis=-1)

    grid = (n // _TILE,)
    h_out, small_out = pl.pallas_call(
        _tile_body,
        grid=grid,
        in_specs=[
            pl.BlockSpec((_TILE, 1), lambda t: (t, 0)),
            pl.BlockSpec((_TILE, d), lambda t: (t, 0)),
            pl.BlockSpec((_TILE, d), lambda t: (t, 0)),
            pl.BlockSpec((_N_EXP, d, d), lambda t: (0, 0, 0)),
            pl.BlockSpec((_N_EXP, d, d), lambda t: (0, 0, 0)),
            pl.BlockSpec((_N_EXP, d), lambda t: (0, 0)),
            pl.BlockSpec((_N_EXP, d, _SMALL), lambda t: (0, 0, 0)),
            pl.BlockSpec((_N_EXP, _SMALL), lambda t: (0, 0)),
        ],
        out_specs=[
            pl.BlockSpec((_TILE, d), lambda t: (t, 0)),
            pl.BlockSpec((_TILE, _SMALL), lambda t: (t, 0)),
        ],
        out_shape=[
            jax.ShapeDtypeStruct((n, d), jnp.float32),
            jax.ShapeDtypeStruct((n, _SMALL), jnp.float32),
        ],
        compiler_params=pltpu.CompilerParams(
            dimension_semantics=("arbitrary",),
        ),
    )(instrf, base, rnnm, w1, wh, b1s, wc, bc)

    value = small_out[:, 0:1]
    action = small_out[:, 1:2].astype(jnp.int32)
    alp = small_out[:, 2:3]
    return value, action, alp, h_out
